# Initial kernel scaffold; baseline (speedup 1.0000x reference)
import jax
import jax.numpy as jnp
from jax.experimental import pallas as pl


def kernel(outputs, outputs_classifier, labels, weight_bias):
    raise NotImplementedError("write your pallas kernel here")


# speedup vs baseline: 53.9188x; 53.9188x over previous
import functools
import math

import jax
import jax.numpy as jnp
from jax.experimental import pallas as pl
from jax.experimental.pallas import tpu as pltpu

_ALPHA = 0.1
_ARGS_BIAS = 0.0
_ARGS_GAMMA = 0.5
_NEG_INF = float("-inf")


def _loss_kernel(labels_ref, wb_ref, x_ref, cls_ref, loss_ref,
                 m_ref, s_ref, ll_ref, a_ref, b_ref,
                 mc_ref, sc_ref, llc_ref,
                 *, ncol, ncls, blk_c, v, rows, total_b):
    i = pl.program_id(0)
    j = pl.program_id(1)

    @pl.when(j == 0)
    def _init():
        m_ref[...] = jnp.full_like(m_ref, _NEG_INF)
        s_ref[...] = jnp.zeros_like(s_ref)
        ll_ref[...] = jnp.zeros_like(ll_ref)
        a_ref[...] = jnp.full_like(a_ref, _NEG_INF)
        b_ref[...] = jnp.full_like(b_ref, _NEG_INF)
        mc_ref[...] = jnp.full_like(mc_ref, _NEG_INF)
        sc_ref[...] = jnp.zeros_like(sc_ref)
        llc_ref[...] = jnp.zeros_like(llc_ref)

    labels = labels_ref[...]
    col_ids = j * blk_c + jax.lax.broadcasted_iota(jnp.int32, (rows, blk_c), 1)
    valid = col_ids < v
    lab_mask = col_ids == labels

    x = x_ref[...]
    xm = jnp.where(valid, x, _NEG_INF)
    bm = jnp.max(xm, axis=1, keepdims=True)
    m_old = m_ref[...]
    m_new = jnp.maximum(m_old, bm)
    sexp = jnp.sum(jnp.exp(xm - m_new), axis=1, keepdims=True)
    s_ref[...] = s_ref[...] * jnp.exp(m_old - m_new) + sexp
    m_ref[...] = m_new
    ll_ref[...] = ll_ref[...] + jnp.sum(
        jnp.where(lab_mask, x, 0.0), axis=1, keepdims=True)

    eq = xm == bm
    strict = jnp.max(jnp.where(eq, _NEG_INF, xm), axis=1, keepdims=True)
    n0 = jnp.sum(jnp.where(eq, 1.0, 0.0), axis=1, keepdims=True)
    c1 = jnp.where(n0 > 1.0, bm, strict)
    a_old = a_ref[...]
    a_ref[...] = jnp.maximum(a_old, bm)
    b_ref[...] = jnp.maximum(jnp.minimum(a_old, bm),
                             jnp.maximum(b_ref[...], c1))

    for k in range(ncls):
        xc = cls_ref[k]
        xcm = jnp.where(valid, xc, _NEG_INF)
        bmc = jnp.max(xcm, axis=1, keepdims=True)
        mc_old = mc_ref[k]
        mc_new = jnp.maximum(mc_old, bmc)
        sexpc = jnp.sum(jnp.exp(xcm - mc_new), axis=1, keepdims=True)
        sc_ref[k] = sc_ref[k] * jnp.exp(mc_old - mc_new) + sexpc
        mc_ref[k] = mc_new
        llc_ref[k] = llc_ref[k] + jnp.sum(
            jnp.where(lab_mask, xc, 0.0), axis=1, keepdims=True)

    @pl.when(j == ncol - 1)
    def _fin():
        th1 = wb_ref[0]
        th2 = wb_ref[1]
        bb = wb_ref[2]
        logz = m_ref[...] + jnp.log(s_ref[...])
        xg = ll_ref[...]
        ce = jnp.sum(logz - xg)
        for k in range(ncls):
            logzc = mc_ref[k] + jnp.log(sc_ref[k])
            ce = ce + jnp.sum(logzc - llc_ref[k])
        t0 = a_ref[...]
        t1 = b_ref[...]
        y = jnp.where(t0 == xg, t1, jnp.where(t1 == xg, t0, t0 + t1))
        dist = (th1 * xg + th2 * y + bb - _ARGS_BIAS) / jnp.sqrt(
            th1 * th1 + th2 * th2)
        per = jnp.where(dist >= 10.0, -2.0,
                        jnp.where(dist >= 0.0, -_ARGS_GAMMA * dist, -dist))
        block_loss = ce / total_b + _ALPHA * jnp.sum(per)

        @pl.when(i == 0)
        def _first():
            loss_ref[0, 0] = block_loss

        @pl.when(i > 0)
        def _rest():
            loss_ref[0, 0] = loss_ref[0, 0] + block_loss


def kernel(outputs, outputs_classifier, labels, weight_bias):
    bn, vn = outputs.shape
    ncls = outputs_classifier.shape[0]
    rows = 256 if bn % 256 == 0 else bn
    blk_c = min(6272, ((vn + 127) // 128) * 128)
    nrb = bn // rows
    ncol = math.ceil(vn / blk_c)

    labels2d = labels[:, None]

    out = pl.pallas_call(
        functools.partial(_loss_kernel, ncol=ncol, ncls=ncls, blk_c=blk_c,
                          v=vn, rows=rows, total_b=bn),
        grid=(nrb, ncol),
        in_specs=[
            pl.BlockSpec((rows, 1), lambda i, j: (i, 0)),
            pl.BlockSpec(memory_space=pltpu.SMEM),
            pl.BlockSpec((rows, blk_c), lambda i, j: (i, j)),
            pl.BlockSpec((ncls, rows, blk_c), lambda i, j: (0, i, j)),
        ],
        out_specs=pl.BlockSpec((1, 1), lambda i, j: (0, 0),
                               memory_space=pltpu.SMEM),
        out_shape=jax.ShapeDtypeStruct((1, 1), jnp.float32),
        scratch_shapes=[
            pltpu.VMEM((rows, 1), jnp.float32),
            pltpu.VMEM((rows, 1), jnp.float32),
            pltpu.VMEM((rows, 1), jnp.float32),
            pltpu.VMEM((rows, 1), jnp.float32),
            pltpu.VMEM((rows, 1), jnp.float32),
            pltpu.VMEM((ncls, rows, 1), jnp.float32),
            pltpu.VMEM((ncls, rows, 1), jnp.float32),
            pltpu.VMEM((ncls, rows, 1), jnp.float32),
        ],
    )(labels2d, weight_bias, outputs, outputs_classifier)
    return out[0, 0]
